# baseline (speedup 1.0000x reference)
import math
import jax
import jax.numpy as jnp
from jax.experimental import pallas as pl
from jax.experimental.pallas import tpu as pltpu

_N = 1000
_NPAD = 1024
_C = 81
_CM1 = 80
_IMG_W = 1024.0
_IMG_H = 1024.0
_SCORE_THRESH = 0.05
_NMS_THRESH = 0.5
_DETS = 100
_WX, _WY, _WW, _WH = 10.0, 10.0, 5.0, 5.0
_CLIP = math.log(1000.0 / 16.0)
_BR = 128
_NB = _NPAD // _BR


def _dense_body(logits_ref, dx_ref, dy_ref, dw_ref, dh_ref, pb_ref,
                probs_ref, x1_ref, y1_ref, x2_ref, y2_ref):
    logits = logits_ref[...]
    m = jnp.max(logits, axis=1, keepdims=True)
    e = jnp.exp(logits - m)
    probs = e / jnp.sum(e, axis=1, keepdims=True)
    probs_ref[...] = probs[:, 1:]

    pb = pb_ref[...]
    widths = pb[:, 2:3] - pb[:, 0:1] + 1.0
    heights = pb[:, 3:4] - pb[:, 1:2] + 1.0
    ctr_x = pb[:, 0:1] + 0.5 * widths
    ctr_y = pb[:, 1:2] + 0.5 * heights

    dx = dx_ref[...] / _WX
    dy = dy_ref[...] / _WY
    dw = jnp.minimum(dw_ref[...] / _WW, _CLIP)
    dh = jnp.minimum(dh_ref[...] / _WH, _CLIP)

    pred_ctr_x = dx * widths + ctr_x
    pred_ctr_y = dy * heights + ctr_y
    pred_w = jnp.exp(dw) * widths
    pred_h = jnp.exp(dh) * heights

    x1 = pred_ctr_x - 0.5 * pred_w
    y1 = pred_ctr_y - 0.5 * pred_h
    x2 = pred_ctr_x + 0.5 * pred_w - 1.0
    y2 = pred_ctr_y + 0.5 * pred_h - 1.0

    x1_ref[...] = jnp.clip(x1, 0.0, _IMG_W - 1.0)
    y1_ref[...] = jnp.clip(y1, 0.0, _IMG_H - 1.0)
    x2_ref[...] = jnp.clip(x2, 0.0, _IMG_W - 1.0)
    y2_ref[...] = jnp.clip(y2, 0.0, _IMG_H - 1.0)


def _dense_stage(class_logits, box_regression, proposal_boxes):
    rc = box_regression.reshape(_N, _C, 4)[:, 1:]
    out_shapes = tuple(
        jax.ShapeDtypeStruct((_N, _CM1), jnp.float32) for _ in range(5))
    return pl.pallas_call(
        _dense_body,
        out_shape=out_shapes,
    )(class_logits, rc[:, :, 0], rc[:, :, 1], rc[:, :, 2], rc[:, :, 3],
      proposal_boxes)


def _lane_scalar(row, c):
    lanes = jax.lax.broadcasted_iota(jnp.int32, (1, _CM1), 1)
    return jnp.sum(jnp.where(lanes == c, row, 0.0), axis=1, keepdims=True)


def _nms_topk_body(x1_ref, y1_ref, x2_ref, y2_ref, s_ref,
                   ob_ref, os_ref, ol_ref,
                   keep_ref, area_ref, key_ref):
    s = s_ref[...]
    valid = s > _SCORE_THRESH
    keep_ref[...] = valid.astype(jnp.float32)
    x1 = x1_ref[...]
    y1 = y1_ref[...]
    x2 = x2_ref[...]
    y2 = y2_ref[...]
    area_ref[...] = (x2 - x1 + 1.0) * (y2 - y1 + 1.0)

    row_any = jnp.any(valid, axis=1, keepdims=True)
    t_last = jnp.sum(row_any.astype(jnp.int32))
    bmax = (t_last + _BR - 1) // _BR

    def step(i, carry):
        krow = keep_ref[pl.ds(i, 1), :]

        @pl.when(jnp.max(krow) > 0.0)
        def _():
            x1i = x1_ref[pl.ds(i, 1), :]
            y1i = y1_ref[pl.ds(i, 1), :]
            x2i = x2_ref[pl.ds(i, 1), :]
            y2i = y2_ref[pl.ds(i, 1), :]
            ai = area_ref[pl.ds(i, 1), :]

            def blk(b, c2):
                off = b * _BR
                x1b = x1_ref[pl.ds(off, _BR), :]
                y1b = y1_ref[pl.ds(off, _BR), :]
                x2b = x2_ref[pl.ds(off, _BR), :]
                y2b = y2_ref[pl.ds(off, _BR), :]
                ab = area_ref[pl.ds(off, _BR), :]
                xx1 = jnp.maximum(x1i, x1b)
                yy1 = jnp.maximum(y1i, y1b)
                xx2 = jnp.minimum(x2i, x2b)
                yy2 = jnp.minimum(y2i, y2b)
                w = jnp.maximum(xx2 - xx1 + 1.0, 0.0)
                h = jnp.maximum(yy2 - yy1 + 1.0, 0.0)
                inter = w * h
                iou = inter / (ai + ab - inter)
                rows = off + jax.lax.broadcasted_iota(
                    jnp.int32, (_BR, _CM1), 0)
                sup = (iou > _NMS_THRESH) & (rows > i) & (krow > 0.0)
                kb = keep_ref[pl.ds(off, _BR), :]
                keep_ref[pl.ds(off, _BR), :] = jnp.where(sup, 0.0, kb)
                return c2

            jax.lax.fori_loop(i // _BR, bmax, blk, 0)
        return carry

    jax.lax.fori_loop(0, t_last, step, 0)
    key_ref[...] = jnp.where(keep_ref[...] > 0.0, s, -1.0)

    lanes1 = jax.lax.broadcasted_iota(jnp.int32, (1, _CM1), 1)

    def pick(p, carry):
        def mblk(b, acc):
            blk = key_ref[pl.ds(b * _BR, _BR), :]
            return jnp.maximum(acc, jnp.max(blk))
        m = jax.lax.fori_loop(0, bmax, mblk, jnp.float32(-3.0))

        @pl.when(m > 0.0)
        def _():
            def cblk(b, acc):
                blk = key_ref[pl.ds(b * _BR, _BR), :]
                hit = jnp.max(jnp.where(blk == m, 1.0, 0.0), axis=0,
                              keepdims=True)
                return jnp.maximum(acc, hit)
            chit = jax.lax.fori_loop(
                0, bmax, cblk, jnp.zeros((1, _CM1), jnp.float32))
            c = jnp.min(jnp.where(chit > 0.0, lanes1, 10_000))

            def rblk(b, acc):
                off = b * _BR
                blk = key_ref[pl.ds(off, _BR), :]
                rows = off + jax.lax.broadcasted_iota(
                    jnp.int32, (_BR, _CM1), 0)
                lanes = jax.lax.broadcasted_iota(
                    jnp.int32, (_BR, _CM1), 1)
                cand = jnp.where((blk == m) & (lanes == c), rows, 1_000_000)
                return jnp.minimum(acc, jnp.min(cand))
            r = jax.lax.fori_loop(0, bmax, rblk, jnp.int32(1_000_000))

            krow = key_ref[pl.ds(r, 1), :]
            key_ref[pl.ds(r, 1), :] = jnp.where(lanes1 == c, -2.0, krow)

            ob_ref[pl.ds(p, 1), pl.ds(0, 1)] = _lane_scalar(x1_ref[pl.ds(r, 1), :], c)
            ob_ref[pl.ds(p, 1), pl.ds(1, 1)] = _lane_scalar(y1_ref[pl.ds(r, 1), :], c)
            ob_ref[pl.ds(p, 1), pl.ds(2, 1)] = _lane_scalar(x2_ref[pl.ds(r, 1), :], c)
            ob_ref[pl.ds(p, 1), pl.ds(3, 1)] = _lane_scalar(y2_ref[pl.ds(r, 1), :], c)
            os_ref[pl.ds(p, 1), pl.ds(0, 1)] = jnp.reshape(m, (1, 1))
            ol_ref[pl.ds(p, 1), pl.ds(0, 1)] = jnp.reshape(c + 1, (1, 1))

        @pl.when(m <= 0.0)
        def _():
            def rblk2(b, acc):
                off = b * _BR
                blk = key_ref[pl.ds(off, _BR), :]
                rows = off + jax.lax.broadcasted_iota(
                    jnp.int32, (_BR, _CM1), 0)
                lanes = jax.lax.broadcasted_iota(
                    jnp.int32, (_BR, _CM1), 1)
                cand = jnp.where(
                    (blk == -1.0) & (lanes == 0) & (rows < _N),
                    rows, 1_000_000)
                return jnp.minimum(acc, jnp.min(cand))
            r = jax.lax.fori_loop(0, _NB, rblk2, jnp.int32(1_000_000))

            krow = key_ref[pl.ds(r, 1), :]
            key_ref[pl.ds(r, 1), :] = jnp.where(lanes1 == 0, -2.0, krow)

            ob_ref[pl.ds(p, 1), pl.ds(0, 1)] = _lane_scalar(x1_ref[pl.ds(r, 1), :], 0)
            ob_ref[pl.ds(p, 1), pl.ds(1, 1)] = _lane_scalar(y1_ref[pl.ds(r, 1), :], 0)
            ob_ref[pl.ds(p, 1), pl.ds(2, 1)] = _lane_scalar(x2_ref[pl.ds(r, 1), :], 0)
            ob_ref[pl.ds(p, 1), pl.ds(3, 1)] = _lane_scalar(y2_ref[pl.ds(r, 1), :], 0)
            os_ref[pl.ds(p, 1), pl.ds(0, 1)] = jnp.zeros((1, 1), jnp.float32)
            ol_ref[pl.ds(p, 1), pl.ds(0, 1)] = jnp.zeros((1, 1), jnp.int32)

        return carry

    jax.lax.fori_loop(0, _DETS, pick, 0)


def _nms_topk_stage(x1s, y1s, x2s, y2s, s_s):
    return pl.pallas_call(
        _nms_topk_body,
        out_shape=(
            jax.ShapeDtypeStruct((_DETS, 4), jnp.float32),
            jax.ShapeDtypeStruct((_DETS, 1), jnp.float32),
            jax.ShapeDtypeStruct((_DETS, 1), jnp.int32),
        ),
        scratch_shapes=[
            pltpu.VMEM((_NPAD, _CM1), jnp.float32),
            pltpu.VMEM((_NPAD, _CM1), jnp.float32),
            pltpu.VMEM((_NPAD, _CM1), jnp.float32),
        ],
    )(x1s, y1s, x2s, y2s, s_s)


def kernel(class_logits, box_regression, proposal_boxes):
    probs, x1, y1, x2, y2 = _dense_stage(
        class_logits, box_regression, proposal_boxes)

    nk, x1s, y1s, x2s, y2s = jax.lax.sort(
        (-probs, x1, y1, x2, y2), dimension=0, is_stable=True, num_keys=1)
    s_s = -nk

    pad = ((0, _NPAD - _N), (0, 0))
    ob, osc, ol = _nms_topk_stage(
        jnp.pad(x1s, pad), jnp.pad(y1s, pad), jnp.pad(x2s, pad),
        jnp.pad(y2s, pad), jnp.pad(s_s, pad, constant_values=-1.0))
    return ob, osc.reshape(-1), ol.reshape(-1)


# speedup vs baseline: 1.0074x; 1.0074x over previous
import math
import jax
import jax.numpy as jnp
from jax.experimental import pallas as pl
from jax.experimental.pallas import tpu as pltpu

_N = 1000
_NPAD = 1024
_C = 81
_CM1 = 80
_IMG_W = 1024.0
_IMG_H = 1024.0
_SCORE_THRESH = 0.05
_NMS_THRESH = 0.5
_DETS = 100
_WX, _WY, _WW, _WH = 10.0, 10.0, 5.0, 5.0
_CLIP = math.log(1000.0 / 16.0)
_BR = 128
_NB = _NPAD // _BR


def _dense_body(logits_ref, dx_ref, dy_ref, dw_ref, dh_ref, pb_ref,
                probs_ref, x1_ref, y1_ref, x2_ref, y2_ref):
    logits = logits_ref[...]
    m = jnp.max(logits, axis=1, keepdims=True)
    e = jnp.exp(logits - m)
    probs = e / jnp.sum(e, axis=1, keepdims=True)
    probs_ref[...] = probs[:, 1:]

    pb = pb_ref[...]
    widths = pb[:, 2:3] - pb[:, 0:1] + 1.0
    heights = pb[:, 3:4] - pb[:, 1:2] + 1.0
    ctr_x = pb[:, 0:1] + 0.5 * widths
    ctr_y = pb[:, 1:2] + 0.5 * heights

    dx = dx_ref[...] / _WX
    dy = dy_ref[...] / _WY
    dw = jnp.minimum(dw_ref[...] / _WW, _CLIP)
    dh = jnp.minimum(dh_ref[...] / _WH, _CLIP)

    pred_ctr_x = dx * widths + ctr_x
    pred_ctr_y = dy * heights + ctr_y
    pred_w = jnp.exp(dw) * widths
    pred_h = jnp.exp(dh) * heights

    x1 = pred_ctr_x - 0.5 * pred_w
    y1 = pred_ctr_y - 0.5 * pred_h
    x2 = pred_ctr_x + 0.5 * pred_w - 1.0
    y2 = pred_ctr_y + 0.5 * pred_h - 1.0

    x1_ref[...] = jnp.clip(x1, 0.0, _IMG_W - 1.0)
    y1_ref[...] = jnp.clip(y1, 0.0, _IMG_H - 1.0)
    x2_ref[...] = jnp.clip(x2, 0.0, _IMG_W - 1.0)
    y2_ref[...] = jnp.clip(y2, 0.0, _IMG_H - 1.0)


def _dense_stage(class_logits, box_regression, proposal_boxes):
    rc = box_regression.reshape(_N, _C, 4)[:, 1:]
    out_shapes = tuple(
        jax.ShapeDtypeStruct((_N, _CM1), jnp.float32) for _ in range(5))
    return pl.pallas_call(
        _dense_body,
        out_shape=out_shapes,
    )(class_logits, rc[:, :, 0], rc[:, :, 1], rc[:, :, 2], rc[:, :, 3],
      proposal_boxes)


def _lane_scalar(row, c):
    lanes = jax.lax.broadcasted_iota(jnp.int32, (1, _CM1), 1)
    return jnp.sum(jnp.where(lanes == c, row, 0.0), axis=1, keepdims=True)


def _nms_topk_body(x1_ref, y1_ref, x2_ref, y2_ref, s_ref,
                   ob_ref, os_ref, ol_ref,
                   keep_ref, area_ref, key_ref):
    s = s_ref[...]
    valid = s > _SCORE_THRESH
    keep_ref[...] = valid.astype(jnp.float32)
    x1 = x1_ref[...]
    y1 = y1_ref[...]
    x2 = x2_ref[...]
    y2 = y2_ref[...]
    area_ref[...] = (x2 - x1 + 1.0) * (y2 - y1 + 1.0)

    row_any = jnp.any(valid, axis=1, keepdims=True)
    t_last = jnp.sum(row_any.astype(jnp.int32))
    bmax = (t_last + _BR - 1) // _BR

    def step(i, carry):
        krow = keep_ref[pl.ds(i, 1), :]

        @pl.when(jnp.max(krow) > 0.0)
        def _():
            x1i = x1_ref[pl.ds(i, 1), :]
            y1i = y1_ref[pl.ds(i, 1), :]
            x2i = x2_ref[pl.ds(i, 1), :]
            y2i = y2_ref[pl.ds(i, 1), :]
            ai = area_ref[pl.ds(i, 1), :]

            def blk(b, c2):
                off = b * _BR
                x1b = x1_ref[pl.ds(off, _BR), :]
                y1b = y1_ref[pl.ds(off, _BR), :]
                x2b = x2_ref[pl.ds(off, _BR), :]
                y2b = y2_ref[pl.ds(off, _BR), :]
                ab = area_ref[pl.ds(off, _BR), :]
                xx1 = jnp.maximum(x1i, x1b)
                yy1 = jnp.maximum(y1i, y1b)
                xx2 = jnp.minimum(x2i, x2b)
                yy2 = jnp.minimum(y2i, y2b)
                w = jnp.maximum(xx2 - xx1 + 1.0, 0.0)
                h = jnp.maximum(yy2 - yy1 + 1.0, 0.0)
                inter = w * h
                iou = inter / (ai + ab - inter)
                rows = off + jax.lax.broadcasted_iota(
                    jnp.int32, (_BR, _CM1), 0)
                sup = (iou > _NMS_THRESH) & (rows > i) & (krow > 0.0)
                kb = keep_ref[pl.ds(off, _BR), :]
                keep_ref[pl.ds(off, _BR), :] = jnp.where(sup, 0.0, kb)
                return c2

            jax.lax.fori_loop(i // _BR, bmax, blk, 0)
        return carry

    jax.lax.fori_loop(0, t_last, step, 0)
    key_ref[...] = jnp.where(keep_ref[...] > 0.0, s, -1.0)

    lanes1 = jax.lax.broadcasted_iota(jnp.int32, (1, _CM1), 1)

    def cminit(b, acc):
        blk = key_ref[pl.ds(b * _BR, _BR), :]
        return jnp.maximum(acc, jnp.max(blk, axis=0, keepdims=True))
    colmax0 = jax.lax.fori_loop(
        0, bmax, cminit, jnp.full((1, _CM1), -3.0, jnp.float32))

    def pick(p, colmax):
        m = jnp.max(colmax)

        def normal(colmax):
            c = jnp.min(jnp.where(colmax == m, lanes1, 10_000))

            def rblk(b, acc):
                off = b * _BR
                blk = key_ref[pl.ds(off, _BR), :]
                rows = off + jax.lax.broadcasted_iota(
                    jnp.int32, (_BR, _CM1), 0)
                lanes = jax.lax.broadcasted_iota(
                    jnp.int32, (_BR, _CM1), 1)
                cand = jnp.where((blk == m) & (lanes == c), rows, 1_000_000)
                return jnp.minimum(acc, jnp.min(cand))
            r = jax.lax.fori_loop(0, bmax, rblk, jnp.int32(1_000_000))

            krow = key_ref[pl.ds(r, 1), :]
            key_ref[pl.ds(r, 1), :] = jnp.where(lanes1 == c, -2.0, krow)

            def cblk(b, acc):
                blk = key_ref[pl.ds(b * _BR, _BR), :]
                return jnp.maximum(acc, jnp.max(blk, axis=0, keepdims=True))
            newcol = jax.lax.fori_loop(
                0, bmax, cblk, jnp.full((1, _CM1), -3.0, jnp.float32))

            ob_ref[pl.ds(p, 1), pl.ds(0, 1)] = _lane_scalar(x1_ref[pl.ds(r, 1), :], c)
            ob_ref[pl.ds(p, 1), pl.ds(1, 1)] = _lane_scalar(y1_ref[pl.ds(r, 1), :], c)
            ob_ref[pl.ds(p, 1), pl.ds(2, 1)] = _lane_scalar(x2_ref[pl.ds(r, 1), :], c)
            ob_ref[pl.ds(p, 1), pl.ds(3, 1)] = _lane_scalar(y2_ref[pl.ds(r, 1), :], c)
            os_ref[pl.ds(p, 1), pl.ds(0, 1)] = jnp.reshape(m, (1, 1))
            ol_ref[pl.ds(p, 1), pl.ds(0, 1)] = jnp.reshape(c + 1, (1, 1))
            return jnp.where(lanes1 == c, newcol, colmax)

        def fallback(colmax):
            def rblk2(b, acc):
                off = b * _BR
                blk = key_ref[pl.ds(off, _BR), :]
                rows = off + jax.lax.broadcasted_iota(
                    jnp.int32, (_BR, _CM1), 0)
                lanes = jax.lax.broadcasted_iota(
                    jnp.int32, (_BR, _CM1), 1)
                cand = jnp.where(
                    (blk == -1.0) & (lanes == 0) & (rows < _N),
                    rows, 1_000_000)
                return jnp.minimum(acc, jnp.min(cand))
            r = jax.lax.fori_loop(0, _NB, rblk2, jnp.int32(1_000_000))

            krow = key_ref[pl.ds(r, 1), :]
            key_ref[pl.ds(r, 1), :] = jnp.where(lanes1 == 0, -2.0, krow)

            ob_ref[pl.ds(p, 1), pl.ds(0, 1)] = _lane_scalar(x1_ref[pl.ds(r, 1), :], 0)
            ob_ref[pl.ds(p, 1), pl.ds(1, 1)] = _lane_scalar(y1_ref[pl.ds(r, 1), :], 0)
            ob_ref[pl.ds(p, 1), pl.ds(2, 1)] = _lane_scalar(x2_ref[pl.ds(r, 1), :], 0)
            ob_ref[pl.ds(p, 1), pl.ds(3, 1)] = _lane_scalar(y2_ref[pl.ds(r, 1), :], 0)
            os_ref[pl.ds(p, 1), pl.ds(0, 1)] = jnp.zeros((1, 1), jnp.float32)
            ol_ref[pl.ds(p, 1), pl.ds(0, 1)] = jnp.zeros((1, 1), jnp.int32)
            return colmax

        return jax.lax.cond(m > 0.0, normal, fallback, colmax)

    jax.lax.fori_loop(0, _DETS, pick, colmax0)


def _nms_topk_stage(x1s, y1s, x2s, y2s, s_s):
    return pl.pallas_call(
        _nms_topk_body,
        out_shape=(
            jax.ShapeDtypeStruct((_DETS, 4), jnp.float32),
            jax.ShapeDtypeStruct((_DETS, 1), jnp.float32),
            jax.ShapeDtypeStruct((_DETS, 1), jnp.int32),
        ),
        scratch_shapes=[
            pltpu.VMEM((_NPAD, _CM1), jnp.float32),
            pltpu.VMEM((_NPAD, _CM1), jnp.float32),
            pltpu.VMEM((_NPAD, _CM1), jnp.float32),
        ],
    )(x1s, y1s, x2s, y2s, s_s)


def kernel(class_logits, box_regression, proposal_boxes):
    probs, x1, y1, x2, y2 = _dense_stage(
        class_logits, box_regression, proposal_boxes)

    nk, x1s, y1s, x2s, y2s = jax.lax.sort(
        (-probs, x1, y1, x2, y2), dimension=0, is_stable=True, num_keys=1)
    s_s = -nk

    pad = ((0, _NPAD - _N), (0, 0))
    ob, osc, ol = _nms_topk_stage(
        jnp.pad(x1s, pad), jnp.pad(y1s, pad), jnp.pad(x2s, pad),
        jnp.pad(y2s, pad), jnp.pad(s_s, pad, constant_values=-1.0))
    return ob, osc.reshape(-1), ol.reshape(-1)
